# baseline (speedup 1.0000x reference)
import functools

import jax
import jax.numpy as jnp
from jax import lax
from jax.experimental import pallas as pl
from jax.experimental.pallas import tpu as pltpu
from jax.experimental.pallas import tpu_sc as plsc

HIDDEN = 32
BATCH = 4096

_INFO = plsc.get_sparse_core_info()
_NC = _INFO.num_cores
_NS = _INFO.num_subcores
_L = _INFO.num_lanes
_NW = _NC * _NS
_BPW = BATCH // _NW
_NBLK = _BPW // _L


def _score_kernel(hidx_hbm, ridx_hbm, tidx_hbm, ent_hbm, rel_hbm, out_hbm,
                  idx_h, idx_r, idx_t,
                  head_v, relv_v, tail_v, out_v,
                  sem_h, sem_r, sem_t):
    wid = lax.axis_index("s") * _NC + lax.axis_index("c")
    base = wid * _BPW

    pltpu.sync_copy(hidx_hbm.at[pl.ds(base, _BPW)], idx_h)
    pltpu.sync_copy(ridx_hbm.at[pl.ds(base, _BPW)], idx_r)
    pltpu.sync_copy(tidx_hbm.at[pl.ds(base, _BPW)], idx_t)

    g_h = pltpu.async_copy(ent_hbm.at[idx_h], head_v, sem_h)
    g_r = pltpu.async_copy(rel_hbm.at[idx_r], relv_v, sem_r)
    g_t = pltpu.async_copy(ent_hbm.at[idx_t], tail_v, sem_t)
    g_h.wait()
    g_r.wait()
    g_t.wait()

    iota = lax.iota(jnp.int32, _L)
    perms = [iota ^ (1 << b) for b in range(4)]
    for blk in range(_NBLK):
        rowsum = jnp.zeros((_L,), jnp.float32)
        for r in range(_L):
            row = blk * _L + r
            d0 = (head_v[row, pl.ds(0, _L)]
                  + relv_v[row, pl.ds(0, _L)]
                  - tail_v[row, pl.ds(0, _L)])
            d1 = (head_v[row, pl.ds(_L, _L)]
                  + relv_v[row, pl.ds(_L, _L)]
                  - tail_v[row, pl.ds(_L, _L)])
            acc = jnp.abs(d0) + jnp.abs(d1)
            for p in perms:
                acc = acc + jnp.take(acc, p)
            rowsum = jnp.where(iota == r, acc, rowsum)
        out_v[pl.ds(blk * _L, _L)] = rowsum

    pltpu.sync_copy(out_v, out_hbm.at[pl.ds(base, _BPW)])


@jax.jit
def _scores(hidx, ridx, tidx, entity_embedding, relation_embedding):
    mesh = plsc.VectorSubcoreMesh(core_axis_name="c", subcore_axis_name="s")
    kern = functools.partial(
        pl.kernel,
        mesh=mesh,
        compiler_params=pltpu.CompilerParams(use_tc_tiling_on_sc=False),
        out_type=jax.ShapeDtypeStruct((BATCH,), jnp.float32),
        scratch_types=[
            pltpu.VMEM((_BPW,), jnp.int32),
            pltpu.VMEM((_BPW,), jnp.int32),
            pltpu.VMEM((_BPW,), jnp.int32),
            pltpu.VMEM((_BPW, HIDDEN), jnp.float32),
            pltpu.VMEM((_BPW, HIDDEN), jnp.float32),
            pltpu.VMEM((_BPW, HIDDEN), jnp.float32),
            pltpu.VMEM((_BPW,), jnp.float32),
            pltpu.SemaphoreType.DMA,
            pltpu.SemaphoreType.DMA,
            pltpu.SemaphoreType.DMA,
        ],
    )(_score_kernel)
    return kern(hidx, ridx, tidx, entity_embedding, relation_embedding)


def kernel(sample, entity_embedding, relation_embedding):
    out = _scores(sample[:, 0], sample[:, 1], sample[:, 2],
                  entity_embedding, relation_embedding)
    return out.reshape(BATCH, 1)
